# Initial kernel scaffold; baseline (speedup 1.0000x reference)
import jax
import jax.numpy as jnp
from jax.experimental import pallas as pl


def kernel(x, edge_index, W1, as1, ad1, b1, W2, as2, ad2, b2, W3, as3, ad3, b3):
    raise NotImplementedError("write your pallas kernel here")


# speedup vs baseline: 18.4027x; 18.4027x over previous
import functools

import jax
import jax.numpy as jnp
from jax import lax
from jax.experimental import pallas as pl
from jax.experimental.pallas import tpu as pltpu
from jax.experimental.pallas import tpu_sc as plsc

N = 10000
D = 128
HID = 16
HEADS = 8
NCLS = 40
F3 = 48
NPAD = 10240
HPAD = 8
E = 320000
ETOT = N + E
EPAD = 331776
NW = 32
C2 = 128
C3 = 64
EPT = EPAD // NW
NC2 = EPT // C2
NC3 = EPT // C3


def _mesh():
    return plsc.VectorSubcoreMesh(core_axis_name="c", subcore_axis_name="s")


def _tc_project(xin, inv, bias, W, As, Ad, combine):
    fin, fout = W.shape
    Bn = 512

    def body(p_ref, inv_ref, b_ref, w_ref, as_ref, ad_ref,
             h_ref, ast_ref, adt_ref):
        if combine:
            nh = inv_ref.shape[1]
            outc = fin // nh
            rep = (lax.broadcasted_iota(jnp.int32, (nh, fin), 1) // outc
                   == lax.broadcasted_iota(jnp.int32, (nh, fin), 0)
                   ).astype(jnp.float32)
            invexp = jnp.dot(inv_ref[...], rep,
                             preferred_element_type=jnp.float32)
            x = (p_ref[0] + p_ref[1]) * invexp + b_ref[...]
            x = jnp.where(x > 0.0, x, jnp.exp(x) - 1.0)
        else:
            x = p_ref[...]
        h = jnp.dot(x, w_ref[...], preferred_element_type=jnp.float32)
        h_ref[...] = h
        dn = (((1,), (1,)), ((), ()))
        ast_ref[...] = lax.dot_general(as_ref[...], h, dn,
                                       preferred_element_type=jnp.float32)
        adt_ref[...] = lax.dot_general(ad_ref[...], h, dn,
                                       preferred_element_type=jnp.float32)

    if combine:
        first = pl.BlockSpec((2, Bn, fin), lambda i: (0, i, 0))
        nh = inv.shape[1]
    else:
        first = pl.BlockSpec((Bn, fin), lambda i: (i, 0))
        nh = 1
    return pl.pallas_call(
        body,
        grid=(NPAD // Bn,),
        in_specs=[first,
                  pl.BlockSpec((Bn, nh), lambda i: (i, 0)),
                  pl.BlockSpec((1, fin), lambda i: (0, 0)),
                  pl.BlockSpec((fin, fout), lambda i: (0, 0)),
                  pl.BlockSpec((HPAD, fout), lambda i: (0, 0)),
                  pl.BlockSpec((HPAD, fout), lambda i: (0, 0))],
        out_specs=[pl.BlockSpec((Bn, fout), lambda i: (i, 0)),
                   pl.BlockSpec((HPAD, Bn), lambda i: (0, i)),
                   pl.BlockSpec((HPAD, Bn), lambda i: (0, i))],
        out_shape=[jax.ShapeDtypeStruct((NPAD, fout), jnp.float32),
                   jax.ShapeDtypeStruct((HPAD, NPAD), jnp.float32),
                   jax.ShapeDtypeStruct((HPAD, NPAD), jnp.float32)],
    )(xin, inv, bias, W, As, Ad)


def _sc_logits(srcp, dstp, ast, adt, nheads):
    zs = nheads * NPAD // 16

    @functools.partial(
        pl.kernel, mesh=_mesh(),
        compiler_params=pltpu.CompilerParams(needs_layout_passes=False,
                                             use_tc_tiling_on_sc=False),
        out_type=[jax.ShapeDtypeStruct((nheads, EPAD), jnp.float32),
                  jax.ShapeDtypeStruct((2, nheads * NPAD), jnp.float32)],
        scratch_types=[
            pltpu.VMEM((NPAD,), jnp.float32),
            pltpu.VMEM((NPAD,), jnp.float32),
            pltpu.VMEM((C2,), jnp.int32),
            pltpu.VMEM((C2,), jnp.int32),
            pltpu.VMEM((C2,), jnp.float32),
            pltpu.VMEM((C2,), jnp.int32),
            pltpu.VMEM((zs,), jnp.float32),
            pltpu.VMEM_SHARED((nheads * NPAD,), jnp.float32),
        ])
    def k(src_hbm, dst_hbm, as_hbm, ad_hbm, ex_hbm, den_hbm,
          asbuf, adbuf, srcbuf, dstbuf, exbuf, idxbuf, zbuf, den_sp):
        c = lax.axis_index("c")
        s = lax.axis_index("s")
        wid = c * 16 + s

        def zb(i, _):
            zbuf[pl.ds(i * 16, 16)] = jnp.zeros((16,), jnp.float32)
            return 0
        lax.fori_loop(0, zs // 16, zb, 0)
        pltpu.sync_copy(zbuf, den_sp.at[pl.ds(s * zs, zs)])
        plsc.subcore_barrier()

        ebase = wid * EPT
        for h in range(nheads):
            pltpu.sync_copy(as_hbm.at[h], asbuf)
            pltpu.sync_copy(ad_hbm.at[h], adbuf)

            def chunk(i, _):
                gb = ebase + i * C2
                pltpu.sync_copy(src_hbm.at[pl.ds(gb, C2)], srcbuf)
                pltpu.sync_copy(dst_hbm.at[pl.ds(gb, C2)], dstbuf)
                for j in range(C2 // 16):
                    sl = pl.ds(j * 16, 16)
                    sv = plsc.load_gather(asbuf, [srcbuf[sl]])
                    dv = plsc.load_gather(adbuf, [dstbuf[sl]])
                    al = sv + dv
                    al = jnp.maximum(al, al * 0.2)
                    ex = jnp.exp(al)
                    eid = gb + j * 16 + lax.iota(jnp.int32, 16)
                    ex = jnp.where(eid < ETOT, ex, 0.0)
                    exbuf[sl] = ex
                    idxbuf[sl] = dstbuf[sl] * nheads + h
                pltpu.sync_copy(exbuf, ex_hbm.at[h, pl.ds(gb, C2)])
                pltpu.sync_copy(exbuf, den_sp.at[idxbuf], add=True)
                return 0
            lax.fori_loop(0, NC2, chunk, 0)

        plsc.subcore_barrier()
        pltpu.sync_copy(den_sp.at[pl.ds(s * zs, zs)],
                        den_hbm.at[c, pl.ds(s * zs, zs)])

    return k(srcp, dstp, ast, adt)


def _tc_inv(dparts, nheads):
    rows = nheads * NPAD // 128
    dp = dparts.reshape(2, rows, 128)

    def body(d_ref, o_ref):
        o_ref[...] = 1.0 / (d_ref[0] + d_ref[1] + 1e-16)

    out = pl.pallas_call(
        body, out_shape=jax.ShapeDtypeStruct((rows, 128), jnp.float32))(dp)
    return out.reshape(nheads * NPAD)


def _sc_messages(srcp, dstp, hmat, exmat, nheads, fout):
    outc = fout // nheads
    zr = NPAD // 16

    @functools.partial(
        pl.kernel, mesh=_mesh(),
        compiler_params=pltpu.CompilerParams(needs_layout_passes=False,
                                             use_tc_tiling_on_sc=False),
        out_type=jax.ShapeDtypeStruct((2, NPAD, fout), jnp.float32),
        scratch_types=[
            pltpu.VMEM((C3,), jnp.int32),
            pltpu.VMEM((C3,), jnp.int32),
            pltpu.VMEM((C3, fout), jnp.float32),
            pltpu.VMEM((nheads * C3,), jnp.float32),
            pltpu.VMEM((80, fout), jnp.float32),
            pltpu.VMEM_SHARED((NPAD, fout), jnp.float32),
        ])
    def k(src_hbm, dst_hbm, h_hbm, ex_hbm, out_hbm,
          srcbuf, dstbuf, rows, exbuf, zrows, out_sp):
        c = lax.axis_index("c")
        s = lax.axis_index("s")
        wid = c * 16 + s

        def zb(r, _):
            for f in range(fout // 16):
                zrows[r, pl.ds(f * 16, 16)] = jnp.zeros((16,), jnp.float32)
            return 0
        lax.fori_loop(0, 80, zb, 0)
        for t in range(zr // 80):
            pltpu.sync_copy(zrows, out_sp.at[pl.ds(s * zr + t * 80, 80)])
        plsc.subcore_barrier()

        ebase = wid * EPT

        def chunk(i, _):
            gb = ebase + i * C3
            pltpu.sync_copy(src_hbm.at[pl.ds(gb, C3)], srcbuf)
            pltpu.sync_copy(dst_hbm.at[pl.ds(gb, C3)], dstbuf)
            pltpu.sync_copy(h_hbm.at[srcbuf], rows)
            for h in range(nheads):
                pltpu.sync_copy(ex_hbm.at[h, pl.ds(gb, C3)],
                                exbuf.at[pl.ds(h * C3, C3)])

            def srow(j, _):
                for h in range(nheads):
                    av = plsc.load_gather(
                        exbuf, [jnp.full((16,), h * C3, jnp.int32) + j])
                    for q in range(outc // 16):
                        cs = pl.ds(h * outc + q * 16, 16)
                        rows[j, cs] = rows[j, cs] * av
                return 0
            lax.fori_loop(0, C3, srow, 0)
            pltpu.sync_copy(rows, out_sp.at[dstbuf], add=True)
            return 0
        lax.fori_loop(0, NC3, chunk, 0)

        plsc.subcore_barrier()
        for t in range(zr // 80):
            pltpu.sync_copy(out_sp.at[pl.ds(s * zr + t * 80, 80)],
                            out_hbm.at[c, pl.ds(s * zr + t * 80, 80)])

    return k(srcp, dstp, hmat, exmat)


def _tc_final(p3, inv3, b3):
    Bn = 512

    def body(p_ref, inv_ref, b_ref, o_ref):
        o = (p_ref[0] + p_ref[1]) * inv_ref[...]
        o = o[:, :NCLS] + b_ref[...]
        m = jnp.max(o, axis=1, keepdims=True)
        lse = jnp.log(jnp.sum(jnp.exp(o - m), axis=1, keepdims=True)) + m
        o_ref[...] = o - lse

    return pl.pallas_call(
        body,
        grid=(NPAD // Bn,),
        in_specs=[pl.BlockSpec((2, Bn, F3), lambda i: (0, i, 0)),
                  pl.BlockSpec((Bn, 1), lambda i: (i, 0)),
                  pl.BlockSpec((1, NCLS), lambda i: (0, 0))],
        out_specs=pl.BlockSpec((Bn, NCLS), lambda i: (i, 0)),
        out_shape=jax.ShapeDtypeStruct((N, NCLS), jnp.float32),
    )(p3, inv3, b3.reshape(1, NCLS))


def kernel(x, edge_index, W1, as1, ad1, b1, W2, as2, ad2, b2, W3, as3, ad3, b3):
    loops = jnp.arange(N, dtype=jnp.int32)
    padn = EPAD - ETOT
    srcp = jnp.concatenate([edge_index[0], loops,
                            jnp.zeros((padn,), jnp.int32)])
    dstp = jnp.concatenate([edge_index[1], loops,
                            jnp.full((padn,), N, jnp.int32)])

    eye8 = jnp.eye(HEADS, dtype=jnp.float32)
    A1s = (eye8[:, :, None] * as1[0][None, :, :]).reshape(HEADS, D)
    A1d = (eye8[:, :, None] * ad1[0][None, :, :]).reshape(HEADS, D)
    A2s = jnp.zeros((HPAD, D), jnp.float32).at[0].set(as2[0, 0])
    A2d = jnp.zeros((HPAD, D), jnp.float32).at[0].set(ad2[0, 0])
    W3p = jnp.zeros((D, F3), jnp.float32).at[:, :NCLS].set(W3)
    A3s = jnp.zeros((HPAD, F3), jnp.float32).at[0, :NCLS].set(as3[0, 0])
    A3d = jnp.zeros((HPAD, F3), jnp.float32).at[0, :NCLS].set(ad3[0, 0])
    zb = jnp.zeros((1, D), jnp.float32)
    xp = jnp.zeros((NPAD, D), jnp.float32).at[:N].set(x)

    dummy_inv = jnp.zeros((NPAD, 1), jnp.float32)
    h1, ast1, adt1 = _tc_project(xp, dummy_inv, zb, W1, A1s, A1d,
                                 combine=False)
    ex1, dp1 = _sc_logits(srcp, dstp, ast1, adt1, HEADS)
    inv1 = _tc_inv(dp1, HEADS).reshape(NPAD, HEADS)
    p1 = _sc_messages(srcp, dstp, h1, ex1, HEADS, D)

    h2, ast2, adt2 = _tc_project(p1, inv1, b1.reshape(1, D), W2, A2s, A2d,
                                 combine=True)
    ex2, dp2 = _sc_logits(srcp, dstp, ast2, adt2, 1)
    inv2 = _tc_inv(dp2, 1).reshape(NPAD, 1)
    p2 = _sc_messages(srcp, dstp, h2, ex2, 1, D)

    h3, ast3, adt3 = _tc_project(p2, inv2, b2.reshape(1, D), W3p, A3s, A3d,
                                 combine=True)
    ex3, dp3 = _sc_logits(srcp, dstp, ast3, adt3, 1)
    inv3 = _tc_inv(dp3, 1).reshape(NPAD, 1)
    p3 = _sc_messages(srcp, dstp, h3, ex3, 1, F3)

    return _tc_final(p3, inv3, b3)


# speedup vs baseline: 26.0503x; 1.4156x over previous
import functools

import jax
import jax.numpy as jnp
from jax import lax
from jax.experimental import pallas as pl
from jax.experimental.pallas import tpu as pltpu
from jax.experimental.pallas import tpu_sc as plsc

N = 10000
D = 128
HID = 16
HEADS = 8
NCLS = 40
F3 = 48
NPAD = 10240
HPAD = 8
E = 320000
ETOT = N + E
EPAD = 331776
NW = 32
CE = 128
EPT = EPAD // NW
NCH = EPT // CE
ZR = NPAD // 16

_SC_PARAMS = dict(
    compiler_params=pltpu.CompilerParams(needs_layout_passes=False,
                                         use_tc_tiling_on_sc=False))


def _mesh():
    return plsc.VectorSubcoreMesh(core_axis_name="c", subcore_axis_name="s")


def _tc_project(xin, inv, bias, W, M, combine):
    fin, fout = W.shape
    Bn = 512

    def body(p_ref, inv_ref, b_ref, w_ref, m_ref, h_ref, comb_ref):
        if combine:
            nh = inv_ref.shape[1]
            outc = fin // nh
            rep = (lax.broadcasted_iota(jnp.int32, (nh, fin), 1) // outc
                   == lax.broadcasted_iota(jnp.int32, (nh, fin), 0)
                   ).astype(jnp.float32)
            invexp = jnp.dot(inv_ref[...], rep,
                             preferred_element_type=jnp.float32)
            x = (p_ref[0] + p_ref[1]) * invexp + b_ref[...]
            x = jnp.where(x > 0.0, x, jnp.exp(x) - 1.0)
        else:
            x = p_ref[...]
        h = jnp.dot(x, w_ref[...], preferred_element_type=jnp.float32)
        h_ref[...] = h
        comb_ref[...] = jnp.dot(h, m_ref[...],
                                preferred_element_type=jnp.float32)

    if combine:
        first = pl.BlockSpec((2, Bn, fin), lambda i: (0, i, 0))
        nh = inv.shape[1]
    else:
        first = pl.BlockSpec((Bn, fin), lambda i: (i, 0))
        nh = 1
    return pl.pallas_call(
        body,
        grid=(NPAD // Bn,),
        in_specs=[first,
                  pl.BlockSpec((Bn, nh), lambda i: (i, 0)),
                  pl.BlockSpec((1, fin), lambda i: (0, 0)),
                  pl.BlockSpec((fin, fout), lambda i: (0, 0)),
                  pl.BlockSpec((fout, 16), lambda i: (0, 0))],
        out_specs=[pl.BlockSpec((Bn, fout), lambda i: (i, 0)),
                   pl.BlockSpec((Bn, 16), lambda i: (i, 0))],
        out_shape=[jax.ShapeDtypeStruct((NPAD, fout), jnp.float32),
                   jax.ShapeDtypeStruct((NPAD, 16), jnp.float32)],
    )(xin, inv, bias, W, M)


def _sc_logits(srcp, dstp, comb, zden, nheads):
    wide = nheads == 8
    if wide:
        ex_t = jax.ShapeDtypeStruct((EPAD, 16), jnp.float32)
        den_t = jax.ShapeDtypeStruct((2, NPAD, 16), jnp.float32)
        den_sp_t = pltpu.VMEM_SHARED((NPAD, 16), jnp.float32)
        ex_scr = pltpu.VMEM((CE, 16), jnp.float32)
    else:
        ex_t = jax.ShapeDtypeStruct((EPAD,), jnp.float32)
        den_t = jax.ShapeDtypeStruct((2, NPAD), jnp.float32)
        den_sp_t = pltpu.VMEM_SHARED((NPAD,), jnp.float32)
        ex_scr = pltpu.VMEM((CE,), jnp.float32)

    @functools.partial(
        pl.kernel, mesh=_mesh(), **_SC_PARAMS,
        out_type=[ex_t, den_t],
        scratch_types=[
            pltpu.VMEM((CE,), jnp.int32),
            pltpu.VMEM((CE,), jnp.int32),
            pltpu.VMEM((CE, 16), jnp.float32),
            pltpu.VMEM((CE, 16), jnp.float32),
            ex_scr,
            den_sp_t,
        ])
    def k(src_hbm, dst_hbm, comb_hbm, zden_hbm, ex_hbm, den_hbm,
          srcbuf, dstbuf, g1, g2, exv, den_sp):
        c = lax.axis_index("c")
        s = lax.axis_index("s")
        wid = c * 16 + s
        pltpu.sync_copy(zden_hbm, den_sp.at[pl.ds(s * ZR, ZR)])
        plsc.subcore_barrier()

        ebase = wid * EPT
        iot = lax.iota(jnp.int32, 16)
        hcl = jnp.minimum(iot, HEADS - 1)
        hmask = iot < HEADS

        def chunk(i, _):
            gb = ebase + i * CE
            pltpu.sync_copy(src_hbm.at[pl.ds(gb, CE)], srcbuf)
            pltpu.sync_copy(dst_hbm.at[pl.ds(gb, CE)], dstbuf)
            pltpu.sync_copy(comb_hbm.at[srcbuf], g1)
            pltpu.sync_copy(comb_hbm.at[dstbuf], g2)
            if wide:
                def edge(g, _):
                    ev = jnp.full((16,), 0, jnp.int32) + g
                    a = plsc.load_gather(g1, [ev, hcl])
                    b = plsc.load_gather(g2, [ev, hcl + 8])
                    al = a + b
                    al = jnp.maximum(al, al * 0.2)
                    exr = jnp.exp(al)
                    ok = hmask & ((gb + g) < ETOT)
                    exv[g, :] = jnp.where(ok, exr, 0.0)
                    return 0
                lax.fori_loop(0, CE, edge, 0, unroll=4)
            else:
                for g in range(CE // 16):
                    ev = g * 16 + iot
                    a = plsc.load_gather(g1, [ev, jnp.full((16,), 0,
                                                           jnp.int32)])
                    b = plsc.load_gather(g2, [ev, jnp.full((16,), 8,
                                                           jnp.int32)])
                    al = a + b
                    al = jnp.maximum(al, al * 0.2)
                    exr = jnp.exp(al)
                    ok = (gb + ev) < ETOT
                    exv[pl.ds(g * 16, 16)] = jnp.where(ok, exr, 0.0)
            pltpu.sync_copy(exv, ex_hbm.at[pl.ds(gb, CE)])
            pltpu.sync_copy(exv, den_sp.at[dstbuf], add=True)
            return 0
        lax.fori_loop(0, NCH, chunk, 0)

        plsc.subcore_barrier()
        pltpu.sync_copy(den_sp.at[pl.ds(s * ZR, ZR)],
                        den_hbm.at[c, pl.ds(s * ZR, ZR)])

    return k(srcp, dstp, comb, zden)


def _tc_inv(dparts, nheads):
    wide = nheads == 8
    cols = 16 if wide else 1
    rows = NPAD * cols // 128
    dp = dparts.reshape(2, rows, 128)

    def body(d_ref, o_ref):
        o_ref[...] = 1.0 / (d_ref[0] + d_ref[1] + 1e-16)

    out = pl.pallas_call(
        body, out_shape=jax.ShapeDtypeStruct((rows, 128), jnp.float32))(dp)
    out = out.reshape(NPAD, cols)
    return out[:, :HEADS] if wide else out


def _sc_messages(srcp, dstp, hmat, exmat, zout, nheads, fout):
    outc = fout // nheads
    wide = nheads == 8
    ex_scr = pltpu.VMEM((CE, 16), jnp.float32) if wide \
        else pltpu.VMEM((CE,), jnp.float32)

    @functools.partial(
        pl.kernel, mesh=_mesh(), **_SC_PARAMS,
        out_type=jax.ShapeDtypeStruct((2, NPAD, fout), jnp.float32),
        scratch_types=[
            pltpu.VMEM((CE,), jnp.int32),
            pltpu.VMEM((CE,), jnp.int32),
            pltpu.VMEM((CE, fout), jnp.float32),
            ex_scr,
            pltpu.VMEM_SHARED((NPAD, fout), jnp.float32),
        ])
    def k(src_hbm, dst_hbm, h_hbm, ex_hbm, zout_hbm, out_hbm,
          srcbuf, dstbuf, rows, exv, out_sp):
        c = lax.axis_index("c")
        s = lax.axis_index("s")
        wid = c * 16 + s
        pltpu.sync_copy(zout_hbm, out_sp.at[pl.ds(s * ZR, ZR)])
        plsc.subcore_barrier()

        ebase = wid * EPT

        def chunk(i, _):
            gb = ebase + i * CE
            pltpu.sync_copy(src_hbm.at[pl.ds(gb, CE)], srcbuf)
            pltpu.sync_copy(dst_hbm.at[pl.ds(gb, CE)], dstbuf)
            pltpu.sync_copy(h_hbm.at[srcbuf], rows)
            pltpu.sync_copy(ex_hbm.at[pl.ds(gb, CE)], exv)

            def srow(j, _):
                jv = jnp.full((16,), 0, jnp.int32) + j
                if wide:
                    for h in range(nheads):
                        av = plsc.load_gather(
                            exv, [jv, jnp.full((16,), h, jnp.int32)])
                        cs = pl.ds(h * outc, 16)
                        rows[j, cs] = rows[j, cs] * av
                else:
                    av = plsc.load_gather(exv, [jv])
                    for q in range(outc // 16):
                        cs = pl.ds(q * 16, 16)
                        rows[j, cs] = rows[j, cs] * av
                return 0
            lax.fori_loop(0, CE, srow, 0, unroll=4)
            pltpu.sync_copy(rows, out_sp.at[dstbuf], add=True)
            return 0
        lax.fori_loop(0, NCH, chunk, 0)

        plsc.subcore_barrier()
        pltpu.sync_copy(out_sp.at[pl.ds(s * ZR, ZR)],
                        out_hbm.at[c, pl.ds(s * ZR, ZR)])

    return k(srcp, dstp, hmat, exmat, zout)


def _tc_final(p3, inv3, b3):
    Bn = 512

    def body(p_ref, inv_ref, b_ref, o_ref):
        o = (p_ref[0] + p_ref[1]) * inv_ref[...]
        o = o[:, :NCLS] + b_ref[...]
        m = jnp.max(o, axis=1, keepdims=True)
        lse = jnp.log(jnp.sum(jnp.exp(o - m), axis=1, keepdims=True)) + m
        o_ref[...] = o - lse

    return pl.pallas_call(
        body,
        grid=(NPAD // Bn,),
        in_specs=[pl.BlockSpec((2, Bn, F3), lambda i: (0, i, 0)),
                  pl.BlockSpec((Bn, 1), lambda i: (i, 0)),
                  pl.BlockSpec((1, NCLS), lambda i: (0, 0))],
        out_specs=pl.BlockSpec((Bn, NCLS), lambda i: (i, 0)),
        out_shape=jax.ShapeDtypeStruct((N, NCLS), jnp.float32),
    )(p3, inv3, b3.reshape(1, NCLS))


def kernel(x, edge_index, W1, as1, ad1, b1, W2, as2, ad2, b2, W3, as3, ad3, b3):
    loops = jnp.arange(N, dtype=jnp.int32)
    padn = EPAD - ETOT
    srcp = jnp.concatenate([edge_index[0], loops,
                            jnp.zeros((padn,), jnp.int32)])
    dstp = jnp.concatenate([edge_index[1], loops,
                            jnp.full((padn,), N, jnp.int32)])

    eye8 = jnp.eye(HEADS, dtype=jnp.float32)
    A1s = (eye8[:, :, None] * as1[0][None, :, :]).reshape(HEADS, D)
    A1d = (eye8[:, :, None] * ad1[0][None, :, :]).reshape(HEADS, D)
    M1 = jnp.concatenate([A1s.T, A1d.T], axis=1)
    M2 = (jnp.zeros((D, 16), jnp.float32)
          .at[:, 0].set(as2[0, 0]).at[:, 8].set(ad2[0, 0]))
    M3 = (jnp.zeros((F3, 16), jnp.float32)
          .at[:NCLS, 0].set(as3[0, 0]).at[:NCLS, 8].set(ad3[0, 0]))
    W3p = jnp.zeros((D, F3), jnp.float32).at[:, :NCLS].set(W3)
    zb = jnp.zeros((1, D), jnp.float32)
    xp = jnp.zeros((NPAD, D), jnp.float32).at[:N].set(x)
    zden16 = jnp.zeros((ZR, 16), jnp.float32)
    zden1 = jnp.zeros((ZR,), jnp.float32)
    zout128 = jnp.zeros((ZR, D), jnp.float32)
    zout48 = jnp.zeros((ZR, F3), jnp.float32)
    dummy_inv = jnp.zeros((NPAD, 1), jnp.float32)

    h1, comb1 = _tc_project(xp, dummy_inv, zb, W1, M1, combine=False)
    ex1, dp1 = _sc_logits(srcp, dstp, comb1, zden16, HEADS)
    inv1 = _tc_inv(dp1, HEADS)
    p1 = _sc_messages(srcp, dstp, h1, ex1, zout128, HEADS, D)

    h2, comb2 = _tc_project(p1, inv1, b1.reshape(1, D), W2, M2,
                            combine=True)
    ex2, dp2 = _sc_logits(srcp, dstp, comb2, zden1, 1)
    inv2 = _tc_inv(dp2, 1)
    p2 = _sc_messages(srcp, dstp, h2, ex2, zout128, 1, D)

    h3, comb3 = _tc_project(p2, inv2, b2.reshape(1, D), W3p, M3,
                            combine=True)
    ex3, dp3 = _sc_logits(srcp, dstp, comb3, zden1, 1)
    inv3 = _tc_inv(dp3, 1)
    p3 = _sc_messages(srcp, dstp, h3, ex3, zout48, 1, F3)

    return _tc_final(p3, inv3, b3)
